# Initial kernel scaffold; baseline (speedup 1.0000x reference)
import jax
import jax.numpy as jnp
from jax.experimental import pallas as pl


def kernel(h, edge_weight, W1, b1, g1, be1, W2, b2, g2, be2, g3, be3, edge_index):
    raise NotImplementedError("write your pallas kernel here")


# speedup vs baseline: 2.7812x; 2.7812x over previous
import functools

import jax
import jax.numpy as jnp
from jax import lax
from jax.experimental import pallas as pl
from jax.experimental.pallas import tpu as pltpu
from jax.experimental.pallas import tpu_sc as plsc

_N, _E, _D = 10000, 320000, 128
_L = 2
_NC, _NS = 2, 16
_NW = _NC * _NS
_EPW = _E // _NW
_C = 80
_NCH = _EPW // _C
_RPT = 624
_TAIL = _N - _NS * _RPT

_EPS = 1e-5


def _sc_agg_body(h_hbm, zeros_hbm, src_hbm, dst_hbm, w_hbm, out_hbm,
                 src_v, dst_v, w_v, rows_v, agg_sh):
    c = lax.axis_index("c")
    s = lax.axis_index("s")
    pltpu.sync_copy(zeros_hbm.at[pl.ds(s * _RPT, _RPT)],
                    agg_sh.at[pl.ds(s * _RPT, _RPT)])

    @pl.when(s == 0)
    def _():
        pltpu.sync_copy(zeros_hbm.at[pl.ds(_NS * _RPT, _TAIL)],
                        agg_sh.at[pl.ds(_NS * _RPT, _TAIL)])

    plsc.subcore_barrier()

    def chunk(i, carry):
        pltpu.sync_copy(src_hbm.at[c, s, i], src_v)
        pltpu.sync_copy(dst_hbm.at[c, s, i], dst_v)
        pltpu.sync_copy(w_hbm.at[c, s, i], w_v)
        pltpu.sync_copy(h_hbm.at[src_v], rows_v)
        for e in range(_C):
            wb = w_v[e, :]
            for j in range(_D // 16):
                sl = pl.ds(j * 16, 16)
                rows_v[e, sl] = rows_v[e, sl] * wb
        pltpu.sync_copy(rows_v, agg_sh.at[dst_v], add=True)
        return carry

    lax.fori_loop(0, _NCH, chunk, 0)
    plsc.subcore_barrier()
    pltpu.sync_copy(agg_sh.at[pl.ds(s * _RPT, _RPT)],
                    out_hbm.at[c, pl.ds(s * _RPT, _RPT)])

    @pl.when(s == 0)
    def _():
        pltpu.sync_copy(agg_sh.at[pl.ds(_NS * _RPT, _TAIL)],
                        out_hbm.at[c, pl.ds(_NS * _RPT, _TAIL)])


@functools.cache
def _sc_agg():
    return pl.kernel(
        _sc_agg_body,
        out_type=jax.ShapeDtypeStruct((_NC, _N, _D), jnp.float32),
        mesh=plsc.VectorSubcoreMesh(core_axis_name="c", subcore_axis_name="s",
                                    num_cores=_NC, num_subcores=_NS),
        scratch_types=[
            pltpu.VMEM((_C,), jnp.int32),
            pltpu.VMEM((_C,), jnp.int32),
            pltpu.VMEM((_C, 16), jnp.float32),
            pltpu.VMEM((_C, _D), jnp.float32),
            pltpu.VMEM_SHARED((_N, _D), jnp.float32),
        ],
    )


_R = 2000
_NB = _N // _R

_row_spec = pl.BlockSpec((_R, _D), lambda i: (i, 0))
_mat_spec = pl.BlockSpec((_D, _D), lambda i: (0, 0))
_vec_spec = pl.BlockSpec((1, _D), lambda i: (0, 0))
_st_spec = pl.BlockSpec((8, _D), lambda i: (0, 0))
_f32 = jnp.float32


def _accum_stats(i, y, st_ref):
    sv = jnp.sum(y, axis=0, keepdims=True)
    qv = jnp.sum(y * y, axis=0, keepdims=True)
    stv = jnp.concatenate([sv, qv, jnp.zeros((6, _D), _f32)], axis=0)

    @pl.when(i == 0)
    def _():
        st_ref[...] = stv

    @pl.when(i != 0)
    def _():
        st_ref[...] = st_ref[...] + stv


def _bn(x, st, g, be):
    m = st[0:1, :] * (1.0 / _N)
    ex2 = st[1:2, :] * (1.0 / _N)
    v = ex2 - m * m
    return (x - m) * lax.rsqrt(v + _EPS) * g + be


def _k1_body(a0_ref, a1_ref, h_ref, wt_ref, b_ref, y_ref, st_ref):
    i = pl.program_id(0)
    x = a0_ref[...] + a1_ref[...] + h_ref[...]
    y = jnp.dot(x, wt_ref[...], preferred_element_type=_f32) + b_ref[...]
    y_ref[...] = y
    _accum_stats(i, y, st_ref)


def _k2_body(y1_ref, st1_ref, g_ref, be_ref, wt_ref, b_ref, y_ref, st_ref):
    i = pl.program_id(0)
    t = jnp.maximum(_bn(y1_ref[...], st1_ref[...], g_ref[...], be_ref[...]), 0.0)
    y = jnp.dot(t, wt_ref[...], preferred_element_type=_f32) + b_ref[...]
    y_ref[...] = y
    _accum_stats(i, y, st_ref)


def _k3_body(y2_ref, st2_ref, g_ref, be_ref, u_ref, st_ref):
    i = pl.program_id(0)
    u = jnp.maximum(_bn(y2_ref[...], st2_ref[...], g_ref[...], be_ref[...]), 0.0)
    u_ref[...] = u
    _accum_stats(i, u, st_ref)


def _k4_body(u_ref, st3_ref, g_ref, be_ref, h_ref):
    h_ref[...] = jnp.maximum(
        _bn(u_ref[...], st3_ref[...], g_ref[...], be_ref[...]), 0.0)


_k1 = pl.pallas_call(
    _k1_body, grid=(_NB,),
    in_specs=[_row_spec, _row_spec, _row_spec, _mat_spec, _vec_spec],
    out_specs=[_row_spec, _st_spec],
    out_shape=[jax.ShapeDtypeStruct((_N, _D), _f32),
               jax.ShapeDtypeStruct((8, _D), _f32)],
)

_k2 = pl.pallas_call(
    _k2_body, grid=(_NB,),
    in_specs=[_row_spec, _st_spec, _vec_spec, _vec_spec, _mat_spec, _vec_spec],
    out_specs=[_row_spec, _st_spec],
    out_shape=[jax.ShapeDtypeStruct((_N, _D), _f32),
               jax.ShapeDtypeStruct((8, _D), _f32)],
)

_k3 = pl.pallas_call(
    _k3_body, grid=(_NB,),
    in_specs=[_row_spec, _st_spec, _vec_spec, _vec_spec],
    out_specs=[_row_spec, _st_spec],
    out_shape=[jax.ShapeDtypeStruct((_N, _D), _f32),
               jax.ShapeDtypeStruct((8, _D), _f32)],
)

_k4 = pl.pallas_call(
    _k4_body, grid=(_NB,),
    in_specs=[_row_spec, _st_spec, _vec_spec, _vec_spec],
    out_specs=_row_spec,
    out_shape=jax.ShapeDtypeStruct((_N, _D), _f32),
)


def kernel(h, edge_weight, W1, b1, g1, be1, W2, b2, g2, be2, g3, be3,
           edge_index):
    src4 = edge_index[0].reshape(_NC, _NS, _NCH, _C)
    dst4 = edge_index[1].reshape(_NC, _NS, _NCH, _C)
    w4 = edge_weight.reshape(_L, _NC, _NS, _NCH, _C)
    wrep = jnp.broadcast_to(w4[..., None], w4.shape + (16,))
    zeros = jnp.zeros((_N, _D), _f32)
    for l in range(_L):
        agg2 = _sc_agg()(h, zeros, src4, dst4, wrep[l])
        y1, st1 = _k1(agg2[0], agg2[1], h, W1[l].T, b1[l][None, :])
        y2, st2 = _k2(y1, st1, g1[l][None, :], be1[l][None, :],
                      W2[l].T, b2[l][None, :])
        u, st3 = _k3(y2, st2, g2[l][None, :], be2[l][None, :])
        h = _k4(u, st3, g3[l][None, :], be3[l][None, :])
    return h


# speedup vs baseline: 3.9818x; 1.4317x over previous
import functools

import jax
import jax.numpy as jnp
from jax import lax
from jax.experimental import pallas as pl
from jax.experimental.pallas import tpu as pltpu
from jax.experimental.pallas import tpu_sc as plsc

_N, _E, _D = 10000, 320000, 128
_L = 2
_NC, _NS = 2, 16
_NW = _NC * _NS
_EPW = _E // _NW
_C = 80
_NCH = _EPW // _C
_NPAIR = (_NCH + 1) // 2
_RPT = 624
_TAIL = _N - _NS * _RPT

_EPS = 1e-5


def _scale(rows_v, w_pv, slot):
    for g in range(_C // 16):
        wv = w_pv[slot, pl.ds(g * 16, 16)]
        for k in range(16):
            e = g * 16 + k
            wb = wv.at[jnp.full((16,), k, jnp.int32)].get(
                mode="promise_in_bounds")
            for j in range(_D // 16):
                sl = pl.ds(j * 16, 16)
                rows_v[e, sl] = rows_v[e, sl] * wb


def _sc_agg_body(h_hbm, zeros_hbm, prm_hbm, wprm_hbm, out_hbm,
                 p_v, w_pv, rowsA, rowsB, agg_sh, gsemA, gsemB):
    c = lax.axis_index("c")
    s = lax.axis_index("s")
    pltpu.sync_copy(zeros_hbm.at[pl.ds(s * _RPT, _RPT)],
                    agg_sh.at[pl.ds(s * _RPT, _RPT)])

    @pl.when(s == 0)
    def _():
        pltpu.sync_copy(zeros_hbm.at[pl.ds(_NS * _RPT, _TAIL)],
                        agg_sh.at[pl.ds(_NS * _RPT, _TAIL)])

    plsc.subcore_barrier()

    def _do_chunk(rows_v, k):
        _scale(rows_v, w_pv, k)
        pltpu.sync_copy(rows_v, agg_sh.at[p_v.at[1, k]], add=True)

    def pair(j, carry):
        pltpu.sync_copy(prm_hbm.at[c, s, j], p_v)
        pltpu.sync_copy(wprm_hbm.at[c, s, j], w_pv)
        gA = pltpu.make_async_copy(h_hbm.at[p_v.at[0, 0]], rowsA, gsemA)
        gA.start()
        gB = pltpu.make_async_copy(h_hbm.at[p_v.at[0, 1]], rowsB, gsemB)
        gB.start()
        gA.wait()
        _do_chunk(rowsA, 0)
        gB.wait()
        _do_chunk(rowsB, 1)
        return carry

    lax.fori_loop(0, _NPAIR, pair, 0)
    plsc.subcore_barrier()
    pltpu.sync_copy(agg_sh.at[pl.ds(s * _RPT, _RPT)],
                    out_hbm.at[c, pl.ds(s * _RPT, _RPT)])

    @pl.when(s == 0)
    def _():
        pltpu.sync_copy(agg_sh.at[pl.ds(_NS * _RPT, _TAIL)],
                        out_hbm.at[c, pl.ds(_NS * _RPT, _TAIL)])


@functools.cache
def _sc_agg():
    return pl.kernel(
        _sc_agg_body,
        out_type=jax.ShapeDtypeStruct((_NC, _N, _D), jnp.float32),
        mesh=plsc.VectorSubcoreMesh(core_axis_name="c", subcore_axis_name="s",
                                    num_cores=_NC, num_subcores=_NS),
        scratch_types=[
            pltpu.VMEM((2, 2, _C), jnp.int32),
            pltpu.VMEM((2, _C), jnp.float32),
            pltpu.VMEM((_C, _D), jnp.float32),
            pltpu.VMEM((_C, _D), jnp.float32),
            pltpu.VMEM_SHARED((_N, _D), jnp.float32),
            pltpu.SemaphoreType.DMA,
            pltpu.SemaphoreType.DMA,
        ],
    )


_R = 2000
_NB = _N // _R

_row_spec = pl.BlockSpec((_R, _D), lambda i: (i, 0))
_mat_spec = pl.BlockSpec((_D, _D), lambda i: (0, 0))
_vec_spec = pl.BlockSpec((1, _D), lambda i: (0, 0))
_st_spec = pl.BlockSpec((8, _D), lambda i: (0, 0))
_f32 = jnp.float32


def _accum_stats(i, y, st_ref):
    sv = jnp.sum(y, axis=0, keepdims=True)
    qv = jnp.sum(y * y, axis=0, keepdims=True)
    stv = jnp.concatenate([sv, qv, jnp.zeros((6, _D), _f32)], axis=0)

    @pl.when(i == 0)
    def _():
        st_ref[...] = stv

    @pl.when(i != 0)
    def _():
        st_ref[...] = st_ref[...] + stv


def _bn(x, st, g, be):
    m = st[0:1, :] * (1.0 / _N)
    ex2 = st[1:2, :] * (1.0 / _N)
    v = ex2 - m * m
    return (x - m) * lax.rsqrt(v + _EPS) * g + be


def _k1_body(a0_ref, a1_ref, h_ref, wt_ref, b_ref, y_ref, st_ref):
    i = pl.program_id(0)
    x = a0_ref[...] + a1_ref[...] + h_ref[...]
    y = jnp.dot(x, wt_ref[...], preferred_element_type=_f32) + b_ref[...]
    y_ref[...] = y
    _accum_stats(i, y, st_ref)


def _k2_body(y1_ref, st1_ref, g_ref, be_ref, wt_ref, b_ref, y_ref, st_ref):
    i = pl.program_id(0)
    t = jnp.maximum(_bn(y1_ref[...], st1_ref[...], g_ref[...], be_ref[...]), 0.0)
    y = jnp.dot(t, wt_ref[...], preferred_element_type=_f32) + b_ref[...]
    y_ref[...] = y
    _accum_stats(i, y, st_ref)


def _k3_body(y2_ref, st2_ref, g_ref, be_ref, u_ref, st_ref):
    i = pl.program_id(0)
    u = jnp.maximum(_bn(y2_ref[...], st2_ref[...], g_ref[...], be_ref[...]), 0.0)
    u_ref[...] = u
    _accum_stats(i, u, st_ref)


def _k4_body(u_ref, st3_ref, g_ref, be_ref, h_ref):
    h_ref[...] = jnp.maximum(
        _bn(u_ref[...], st3_ref[...], g_ref[...], be_ref[...]), 0.0)


_k1 = pl.pallas_call(
    _k1_body, grid=(_NB,),
    in_specs=[_row_spec, _row_spec, _row_spec, _mat_spec, _vec_spec],
    out_specs=[_row_spec, _st_spec],
    out_shape=[jax.ShapeDtypeStruct((_N, _D), _f32),
               jax.ShapeDtypeStruct((8, _D), _f32)],
)

_k2 = pl.pallas_call(
    _k2_body, grid=(_NB,),
    in_specs=[_row_spec, _st_spec, _vec_spec, _vec_spec, _mat_spec, _vec_spec],
    out_specs=[_row_spec, _st_spec],
    out_shape=[jax.ShapeDtypeStruct((_N, _D), _f32),
               jax.ShapeDtypeStruct((8, _D), _f32)],
)

_k3 = pl.pallas_call(
    _k3_body, grid=(_NB,),
    in_specs=[_row_spec, _st_spec, _vec_spec, _vec_spec],
    out_specs=[_row_spec, _st_spec],
    out_shape=[jax.ShapeDtypeStruct((_N, _D), _f32),
               jax.ShapeDtypeStruct((8, _D), _f32)],
)

_k4 = pl.pallas_call(
    _k4_body, grid=(_NB,),
    in_specs=[_row_spec, _st_spec, _vec_spec, _vec_spec],
    out_specs=_row_spec,
    out_shape=jax.ShapeDtypeStruct((_N, _D), _f32),
)


def kernel(h, edge_weight, W1, b1, g1, be1, W2, b2, g2, be2, g3, be3,
           edge_index):
    def _padc(a):
        return jnp.pad(a, [(0, 0), (0, 0), (0, 2 * _NPAIR - _NCH), (0, 0)]
                       ).reshape(_NC, _NS, _NPAIR, 2, _C)

    src4 = _padc(edge_index[0].reshape(_NC, _NS, _NCH, _C))
    dst4 = _padc(edge_index[1].reshape(_NC, _NS, _NCH, _C))
    w4 = edge_weight.reshape(_L, _NC, _NS, _NCH, _C)
    zeros = jnp.zeros((_N, _D), _f32)
    prm = jnp.stack([src4, dst4], axis=3)
    for l in range(_L):
        wprm = _padc(w4[l])
        agg2 = _sc_agg()(h, zeros, prm, wprm)
        y1, st1 = _k1(agg2[0], agg2[1], h, W1[l].T, b1[l][None, :])
        y2, st2 = _k2(y1, st1, g1[l][None, :], be1[l][None, :],
                      W2[l].T, b2[l][None, :])
        u, st3 = _k3(y2, st2, g2[l][None, :], be2[l][None, :])
        h = _k4(u, st3, g3[l][None, :], be3[l][None, :])
    return h


# speedup vs baseline: 4.1873x; 1.0516x over previous
import functools

import jax
import jax.numpy as jnp
from jax import lax
from jax.experimental import pallas as pl
from jax.experimental.pallas import tpu as pltpu
from jax.experimental.pallas import tpu_sc as plsc

_N, _E, _D = 10000, 320000, 128
_L = 2
_NC, _NS = 2, 16
_NW = _NC * _NS
_EPW = _E // _NW
_C = 80
_NCH = _EPW // _C
_NPAIR = (_NCH + 1) // 2
_RPT = 624
_TAIL = _N - _NS * _RPT

_EPS = 1e-5


def _scale(rows_v, w_pv, slot):
    for g in range(_C // 16):
        wv = w_pv[slot, pl.ds(g * 16, 16)]
        for k in range(16):
            e = g * 16 + k
            wb = wv.at[jnp.full((16,), k, jnp.int32)].get(
                mode="promise_in_bounds")
            for j in range(_D // 16):
                sl = pl.ds(j * 16, 16)
                rows_v[e, sl] = rows_v[e, sl] * wb


def _sc_agg_body(h_hbm, zeros_hbm, prm_hbm, wprm_hbm, out_hbm,
                 p_v, w_pv, rowsA, rowsB, agg_sh, gsemA, gsemB,
                 ssemA, ssemB):
    c = lax.axis_index("c")
    s = lax.axis_index("s")
    pltpu.sync_copy(zeros_hbm.at[pl.ds(s * _RPT, _RPT)],
                    agg_sh.at[pl.ds(s * _RPT, _RPT)])

    @pl.when(s == 0)
    def _():
        pltpu.sync_copy(zeros_hbm.at[pl.ds(_NS * _RPT, _TAIL)],
                        agg_sh.at[pl.ds(_NS * _RPT, _TAIL)])

    plsc.subcore_barrier()

    def pair(j, carry):
        pltpu.sync_copy(prm_hbm.at[c, s, j], p_v)
        pltpu.sync_copy(wprm_hbm.at[c, s, j], w_pv)
        gA = pltpu.make_async_copy(h_hbm.at[p_v.at[0, 0]], rowsA, gsemA)
        gA.start()
        gB = pltpu.make_async_copy(h_hbm.at[p_v.at[0, 1]], rowsB, gsemB)
        gB.start()
        gA.wait()
        _scale(rowsA, w_pv, 0)
        sA = pltpu.make_async_copy(rowsA, agg_sh.at[p_v.at[1, 0]], ssemA)
        sA.start(add=True)
        gB.wait()
        _scale(rowsB, w_pv, 1)
        sB = pltpu.make_async_copy(rowsB, agg_sh.at[p_v.at[1, 1]], ssemB)
        sB.start(add=True)
        sA.wait()
        sB.wait()
        return carry

    lax.fori_loop(0, _NPAIR, pair, 0)
    plsc.subcore_barrier()
    pltpu.sync_copy(agg_sh.at[pl.ds(s * _RPT, _RPT)],
                    out_hbm.at[c, pl.ds(s * _RPT, _RPT)])

    @pl.when(s == 0)
    def _():
        pltpu.sync_copy(agg_sh.at[pl.ds(_NS * _RPT, _TAIL)],
                        out_hbm.at[c, pl.ds(_NS * _RPT, _TAIL)])


@functools.cache
def _sc_agg():
    return pl.kernel(
        _sc_agg_body,
        out_type=jax.ShapeDtypeStruct((_NC, _N, _D), jnp.float32),
        mesh=plsc.VectorSubcoreMesh(core_axis_name="c", subcore_axis_name="s",
                                    num_cores=_NC, num_subcores=_NS),
        scratch_types=[
            pltpu.VMEM((2, 2, _C), jnp.int32),
            pltpu.VMEM((2, _C), jnp.float32),
            pltpu.VMEM((_C, _D), jnp.float32),
            pltpu.VMEM((_C, _D), jnp.float32),
            pltpu.VMEM_SHARED((_N, _D), jnp.float32),
            pltpu.SemaphoreType.DMA,
            pltpu.SemaphoreType.DMA,
            pltpu.SemaphoreType.DMA,
            pltpu.SemaphoreType.DMA,
        ],
    )


_R = 2000
_NB = _N // _R

_row_spec = pl.BlockSpec((_R, _D), lambda i: (i, 0))
_mat_spec = pl.BlockSpec((_D, _D), lambda i: (0, 0))
_vec_spec = pl.BlockSpec((1, _D), lambda i: (0, 0))
_st_spec = pl.BlockSpec((8, _D), lambda i: (0, 0))
_f32 = jnp.float32


def _accum_stats(i, y, st_ref):
    sv = jnp.sum(y, axis=0, keepdims=True)
    qv = jnp.sum(y * y, axis=0, keepdims=True)
    stv = jnp.concatenate([sv, qv, jnp.zeros((6, _D), _f32)], axis=0)

    @pl.when(i == 0)
    def _():
        st_ref[...] = stv

    @pl.when(i != 0)
    def _():
        st_ref[...] = st_ref[...] + stv


def _bn(x, st, g, be):
    m = st[0:1, :] * (1.0 / _N)
    ex2 = st[1:2, :] * (1.0 / _N)
    v = ex2 - m * m
    return (x - m) * lax.rsqrt(v + _EPS) * g + be


def _k1_body(a0_ref, a1_ref, h_ref, wt_ref, b_ref, y_ref, st_ref):
    i = pl.program_id(0)
    x = a0_ref[...] + a1_ref[...] + h_ref[...]
    y = jnp.dot(x, wt_ref[...], preferred_element_type=_f32) + b_ref[...]
    y_ref[...] = y
    _accum_stats(i, y, st_ref)


def _k2_body(y1_ref, st1_ref, g_ref, be_ref, wt_ref, b_ref, y_ref, st_ref):
    i = pl.program_id(0)
    t = jnp.maximum(_bn(y1_ref[...], st1_ref[...], g_ref[...], be_ref[...]), 0.0)
    y = jnp.dot(t, wt_ref[...], preferred_element_type=_f32) + b_ref[...]
    y_ref[...] = y
    _accum_stats(i, y, st_ref)


def _k3_body(y2_ref, st2_ref, g_ref, be_ref, u_ref, st_ref):
    i = pl.program_id(0)
    u = jnp.maximum(_bn(y2_ref[...], st2_ref[...], g_ref[...], be_ref[...]), 0.0)
    u_ref[...] = u
    _accum_stats(i, u, st_ref)


def _k4_body(u_ref, st3_ref, g_ref, be_ref, h_ref):
    h_ref[...] = jnp.maximum(
        _bn(u_ref[...], st3_ref[...], g_ref[...], be_ref[...]), 0.0)


_k1 = pl.pallas_call(
    _k1_body, grid=(_NB,),
    in_specs=[_row_spec, _row_spec, _row_spec, _mat_spec, _vec_spec],
    out_specs=[_row_spec, _st_spec],
    out_shape=[jax.ShapeDtypeStruct((_N, _D), _f32),
               jax.ShapeDtypeStruct((8, _D), _f32)],
)

_k2 = pl.pallas_call(
    _k2_body, grid=(_NB,),
    in_specs=[_row_spec, _st_spec, _vec_spec, _vec_spec, _mat_spec, _vec_spec],
    out_specs=[_row_spec, _st_spec],
    out_shape=[jax.ShapeDtypeStruct((_N, _D), _f32),
               jax.ShapeDtypeStruct((8, _D), _f32)],
)

_k3 = pl.pallas_call(
    _k3_body, grid=(_NB,),
    in_specs=[_row_spec, _st_spec, _vec_spec, _vec_spec],
    out_specs=[_row_spec, _st_spec],
    out_shape=[jax.ShapeDtypeStruct((_N, _D), _f32),
               jax.ShapeDtypeStruct((8, _D), _f32)],
)

_k4 = pl.pallas_call(
    _k4_body, grid=(_NB,),
    in_specs=[_row_spec, _st_spec, _vec_spec, _vec_spec],
    out_specs=_row_spec,
    out_shape=jax.ShapeDtypeStruct((_N, _D), _f32),
)


def kernel(h, edge_weight, W1, b1, g1, be1, W2, b2, g2, be2, g3, be3,
           edge_index):
    def _padc(a):
        return jnp.pad(a, [(0, 0), (0, 0), (0, 2 * _NPAIR - _NCH), (0, 0)]
                       ).reshape(_NC, _NS, _NPAIR, 2, _C)

    src4 = _padc(edge_index[0].reshape(_NC, _NS, _NCH, _C))
    dst4 = _padc(edge_index[1].reshape(_NC, _NS, _NCH, _C))
    w4 = edge_weight.reshape(_L, _NC, _NS, _NCH, _C)
    zeros = jnp.zeros((_N, _D), _f32)
    prm = jnp.stack([src4, dst4], axis=3)
    for l in range(_L):
        wprm = _padc(w4[l])
        agg2 = _sc_agg()(h, zeros, prm, wprm)
        y1, st1 = _k1(agg2[0], agg2[1], h, W1[l].T, b1[l][None, :])
        y2, st2 = _k2(y1, st1, g1[l][None, :], be1[l][None, :],
                      W2[l].T, b2[l][None, :])
        u, st3 = _k3(y2, st2, g2[l][None, :], be2[l][None, :])
        h = _k4(u, st3, g3[l][None, :], be3[l][None, :])
    return h


# speedup vs baseline: 4.2558x; 1.0164x over previous
import functools

import jax
import jax.numpy as jnp
from jax import lax
from jax.experimental import pallas as pl
from jax.experimental.pallas import tpu as pltpu
from jax.experimental.pallas import tpu_sc as plsc

_N, _E, _D = 10000, 320000, 128
_L = 2
_NC, _NS = 2, 16
_NW = _NC * _NS
_EPW = _E // _NW
_C = 80
_NCH = _EPW // _C
_NF = 3
_NGRP = (_NCH + _NF - 1) // _NF
_RPT = 624
_TAIL = _N - _NS * _RPT

_EPS = 1e-5


def _scale(rows_v, w_pv, slot):
    for g in range(_C // 16):
        wv = w_pv[slot, pl.ds(g * 16, 16)]
        for k in range(16):
            e = g * 16 + k
            wb = wv.at[jnp.full((16,), k, jnp.int32)].get(
                mode="promise_in_bounds")
            for j in range(_D // 16):
                sl = pl.ds(j * 16, 16)
                rows_v[e, sl] = rows_v[e, sl] * wb


def _sc_agg_body(h_hbm, zeros_hbm, prm_hbm, wprm_hbm, out_hbm,
                 p_v, w_pv, rowsA, rowsB, rowsC, agg_sh, gsemA, gsemB,
                 gsemC, ssemA, ssemB, ssemC):
    c = lax.axis_index("c")
    s = lax.axis_index("s")
    pltpu.sync_copy(zeros_hbm.at[pl.ds(s * _RPT, _RPT)],
                    agg_sh.at[pl.ds(s * _RPT, _RPT)])

    @pl.when(s == 0)
    def _():
        pltpu.sync_copy(zeros_hbm.at[pl.ds(_NS * _RPT, _TAIL)],
                        agg_sh.at[pl.ds(_NS * _RPT, _TAIL)])

    plsc.subcore_barrier()

    slots = ((rowsA, gsemA, ssemA), (rowsB, gsemB, ssemB),
             (rowsC, gsemC, ssemC))

    def group(j, carry):
        pltpu.sync_copy(prm_hbm.at[c, s, j], p_v)
        pltpu.sync_copy(wprm_hbm.at[c, s, j], w_pv)
        gs = []
        for k, (rows, gsem, _) in enumerate(slots):
            g = pltpu.make_async_copy(h_hbm.at[p_v.at[0, k]], rows, gsem)
            g.start()
            gs.append(g)
        ss = []
        for k, (rows, _, ssem) in enumerate(slots):
            gs[k].wait()
            _scale(rows, w_pv, k)
            sc_ = pltpu.make_async_copy(rows, agg_sh.at[p_v.at[1, k]], ssem)
            sc_.start(add=True)
            ss.append(sc_)
        for sc_ in ss:
            sc_.wait()
        return carry

    lax.fori_loop(0, _NGRP, group, 0)
    plsc.subcore_barrier()
    pltpu.sync_copy(agg_sh.at[pl.ds(s * _RPT, _RPT)],
                    out_hbm.at[c, pl.ds(s * _RPT, _RPT)])

    @pl.when(s == 0)
    def _():
        pltpu.sync_copy(agg_sh.at[pl.ds(_NS * _RPT, _TAIL)],
                        out_hbm.at[c, pl.ds(_NS * _RPT, _TAIL)])


@functools.cache
def _sc_agg():
    return pl.kernel(
        _sc_agg_body,
        out_type=jax.ShapeDtypeStruct((_NC, _N, _D), jnp.float32),
        mesh=plsc.VectorSubcoreMesh(core_axis_name="c", subcore_axis_name="s",
                                    num_cores=_NC, num_subcores=_NS),
        scratch_types=[
            pltpu.VMEM((2, _NF, _C), jnp.int32),
            pltpu.VMEM((_NF, _C), jnp.float32),
            pltpu.VMEM((_C, _D), jnp.float32),
            pltpu.VMEM((_C, _D), jnp.float32),
            pltpu.VMEM((_C, _D), jnp.float32),
            pltpu.VMEM_SHARED((_N, _D), jnp.float32),
            pltpu.SemaphoreType.DMA,
            pltpu.SemaphoreType.DMA,
            pltpu.SemaphoreType.DMA,
            pltpu.SemaphoreType.DMA,
            pltpu.SemaphoreType.DMA,
            pltpu.SemaphoreType.DMA,
        ],
    )


_R = 2000
_NB = _N // _R

_row_spec = pl.BlockSpec((_R, _D), lambda i: (i, 0))
_mat_spec = pl.BlockSpec((_D, _D), lambda i: (0, 0))
_vec_spec = pl.BlockSpec((1, _D), lambda i: (0, 0))
_st_spec = pl.BlockSpec((8, _D), lambda i: (0, 0))
_f32 = jnp.float32


def _accum_stats(i, y, st_ref):
    sv = jnp.sum(y, axis=0, keepdims=True)
    qv = jnp.sum(y * y, axis=0, keepdims=True)
    stv = jnp.concatenate([sv, qv, jnp.zeros((6, _D), _f32)], axis=0)

    @pl.when(i == 0)
    def _():
        st_ref[...] = stv

    @pl.when(i != 0)
    def _():
        st_ref[...] = st_ref[...] + stv


def _bn(x, st, g, be):
    m = st[0:1, :] * (1.0 / _N)
    ex2 = st[1:2, :] * (1.0 / _N)
    v = ex2 - m * m
    return (x - m) * lax.rsqrt(v + _EPS) * g + be


def _k1_body(a0_ref, a1_ref, h_ref, wt_ref, b_ref, y_ref, st_ref):
    i = pl.program_id(0)
    x = a0_ref[...] + a1_ref[...] + h_ref[...]
    y = jnp.dot(x, wt_ref[...], preferred_element_type=_f32) + b_ref[...]
    y_ref[...] = y
    _accum_stats(i, y, st_ref)


def _k2_body(y1_ref, st1_ref, g_ref, be_ref, wt_ref, b_ref, y_ref, st_ref):
    i = pl.program_id(0)
    t = jnp.maximum(_bn(y1_ref[...], st1_ref[...], g_ref[...], be_ref[...]), 0.0)
    y = jnp.dot(t, wt_ref[...], preferred_element_type=_f32) + b_ref[...]
    y_ref[...] = y
    _accum_stats(i, y, st_ref)


def _k3_body(y2_ref, st2_ref, g_ref, be_ref, u_ref, st_ref):
    i = pl.program_id(0)
    u = jnp.maximum(_bn(y2_ref[...], st2_ref[...], g_ref[...], be_ref[...]), 0.0)
    u_ref[...] = u
    _accum_stats(i, u, st_ref)


def _k4_body(u_ref, st3_ref, g_ref, be_ref, h_ref):
    h_ref[...] = jnp.maximum(
        _bn(u_ref[...], st3_ref[...], g_ref[...], be_ref[...]), 0.0)


_k1 = pl.pallas_call(
    _k1_body, grid=(_NB,),
    in_specs=[_row_spec, _row_spec, _row_spec, _mat_spec, _vec_spec],
    out_specs=[_row_spec, _st_spec],
    out_shape=[jax.ShapeDtypeStruct((_N, _D), _f32),
               jax.ShapeDtypeStruct((8, _D), _f32)],
)

_k2 = pl.pallas_call(
    _k2_body, grid=(_NB,),
    in_specs=[_row_spec, _st_spec, _vec_spec, _vec_spec, _mat_spec, _vec_spec],
    out_specs=[_row_spec, _st_spec],
    out_shape=[jax.ShapeDtypeStruct((_N, _D), _f32),
               jax.ShapeDtypeStruct((8, _D), _f32)],
)

_k3 = pl.pallas_call(
    _k3_body, grid=(_NB,),
    in_specs=[_row_spec, _st_spec, _vec_spec, _vec_spec],
    out_specs=[_row_spec, _st_spec],
    out_shape=[jax.ShapeDtypeStruct((_N, _D), _f32),
               jax.ShapeDtypeStruct((8, _D), _f32)],
)

_k4 = pl.pallas_call(
    _k4_body, grid=(_NB,),
    in_specs=[_row_spec, _st_spec, _vec_spec, _vec_spec],
    out_specs=_row_spec,
    out_shape=jax.ShapeDtypeStruct((_N, _D), _f32),
)


def kernel(h, edge_weight, W1, b1, g1, be1, W2, b2, g2, be2, g3, be3,
           edge_index):
    def _padc(a):
        return jnp.pad(a, [(0, 0), (0, 0), (0, _NF * _NGRP - _NCH), (0, 0)]
                       ).reshape(_NC, _NS, _NGRP, _NF, _C)

    src4 = _padc(edge_index[0].reshape(_NC, _NS, _NCH, _C))
    dst4 = _padc(edge_index[1].reshape(_NC, _NS, _NCH, _C))
    w4 = edge_weight.reshape(_L, _NC, _NS, _NCH, _C)
    zeros = jnp.zeros((_N, _D), _f32)
    prm = jnp.stack([src4, dst4], axis=3)
    for l in range(_L):
        wprm = _padc(w4[l])
        agg2 = _sc_agg()(h, zeros, prm, wprm)
        y1, st1 = _k1(agg2[0], agg2[1], h, W1[l].T, b1[l][None, :])
        y2, st2 = _k2(y1, st1, g1[l][None, :], be1[l][None, :],
                      W2[l].T, b2[l][None, :])
        u, st3 = _k3(y2, st2, g2[l][None, :], be2[l][None, :])
        h = _k4(u, st3, g3[l][None, :], be3[l][None, :])
    return h
